# baseline (speedup 1.0000x reference)
import functools

import jax
import jax.numpy as jnp
from jax import lax
from jax.experimental import pallas as pl
from jax.experimental.pallas import tpu as pltpu
from jax.experimental.pallas import tpu_sc as plsc

VOCAB = 1000
VPAD = 1024
BATCH = 4096
BLOCK = 8
N = BATCH * BLOCK
NC = 2
NS = 16
NW = NC * NS
ROWS_PER_W = N // NW
CHUNK = 32
NCHUNK = ROWS_PER_W // CHUNK
NBUF = 4

_mesh = plsc.VectorSubcoreMesh(core_axis_name="c", subcore_axis_name="s")


@functools.partial(
    pl.kernel,
    mesh=_mesh,
    out_type=jax.ShapeDtypeStruct((N, VPAD), jnp.bfloat16),
    scratch_types=[
        pltpu.VMEM((ROWS_PER_W,), jnp.int32),
        pltpu.VMEM((CHUNK, VPAD), jnp.bfloat16),
        pltpu.VMEM((CHUNK, VPAD), jnp.bfloat16),
        pltpu.VMEM((CHUNK, VPAD), jnp.bfloat16),
        pltpu.VMEM((CHUNK, VPAD), jnp.bfloat16),
        pltpu.SemaphoreType.DMA,
        pltpu.SemaphoreType.DMA,
        pltpu.SemaphoreType.DMA,
        pltpu.SemaphoreType.DMA,
        pltpu.SemaphoreType.DMA,
        pltpu.SemaphoreType.DMA,
        pltpu.SemaphoreType.DMA,
        pltpu.SemaphoreType.DMA,
    ],
    compiler_params=pltpu.CompilerParams(use_tc_tiling_on_sc=False),
)
def _gather_kernel(
    w_hbm, idx_hbm, out_hbm, idx_v,
    b0, b1, b2, b3, gs0, gs1, gs2, gs3, ss0, ss1, ss2, ss3,
):
    wid = lax.axis_index("s") * NC + lax.axis_index("c")
    base = wid * ROWS_PER_W
    pltpu.sync_copy(idx_hbm.at[pl.ds(wid * ROWS_PER_W, ROWS_PER_W)], idx_v)
    bufs = (b0, b1, b2, b3)
    gsems = (gs0, gs1, gs2, gs3)
    ssems = (ss0, ss1, ss2, ss3)

    def gather(j):
        slot = j % NBUF
        return pltpu.async_copy(
            w_hbm.at[idx_v.at[pl.ds(j * CHUNK, CHUNK)]], bufs[slot], gsems[slot]
        )

    def scatter(j):
        slot = j % NBUF
        return pltpu.async_copy(
            bufs[slot], out_hbm.at[pl.ds(base + j * CHUNK, CHUNK)], ssems[slot]
        )

    g = [None] * NCHUNK
    s = [None] * NCHUNK
    waited = [False] * NCHUNK
    for j in range(min(NBUF - 1, NCHUNK)):
        g[j] = gather(j)
    for j in range(NCHUNK):
        if j + NBUF - 1 < NCHUNK:
            if j >= 1:
                s[j - 1].wait()
                waited[j - 1] = True
            g[j + NBUF - 1] = gather(j + NBUF - 1)
        g[j].wait()
        s[j] = scatter(j)
    for j in range(NCHUNK):
        if not waited[j]:
            s[j].wait()


def kernel(idx, W):
    w_pad = jnp.pad(W.astype(jnp.bfloat16), ((0, 0), (0, VPAD - VOCAB)))
    flat = idx.reshape(N).astype(jnp.int32)
    out = _gather_kernel(w_pad, flat)
    return out[:, :VOCAB].astype(jnp.float32).reshape(BATCH, BLOCK, VOCAB)


# speedup vs baseline: 1.7320x; 1.7320x over previous
import functools

import jax
import jax.numpy as jnp
from jax import lax
from jax.experimental import pallas as pl
from jax.experimental.pallas import tpu as pltpu
from jax.experimental.pallas import tpu_sc as plsc

VOCAB = 1000
VPAD = 1024
BATCH = 4096
BLOCK = 8
N = BATCH * BLOCK
NC = 2
NS = 16
NW = NC * NS
ROWS_PER_W = N // NW
CHUNK = 16
NCHUNK = ROWS_PER_W // CHUNK
NBUF = 6

_mesh = plsc.VectorSubcoreMesh(core_axis_name="c", subcore_axis_name="s")


@functools.partial(
    pl.kernel,
    mesh=_mesh,
    out_type=jax.ShapeDtypeStruct((N, VPAD), jnp.float32),
    scratch_types=[
        pltpu.VMEM((ROWS_PER_W,), jnp.int32),
        pltpu.VMEM((CHUNK, VPAD), jnp.float32),
        pltpu.VMEM((CHUNK, VPAD), jnp.float32),
        pltpu.VMEM((CHUNK, VPAD), jnp.float32),
        pltpu.VMEM((CHUNK, VPAD), jnp.float32),
        pltpu.VMEM((CHUNK, VPAD), jnp.float32),
        pltpu.VMEM((CHUNK, VPAD), jnp.float32),
        pltpu.SemaphoreType.DMA,
        pltpu.SemaphoreType.DMA,
        pltpu.SemaphoreType.DMA,
        pltpu.SemaphoreType.DMA,
        pltpu.SemaphoreType.DMA,
        pltpu.SemaphoreType.DMA,
        pltpu.SemaphoreType.DMA,
        pltpu.SemaphoreType.DMA,
        pltpu.SemaphoreType.DMA,
        pltpu.SemaphoreType.DMA,
        pltpu.SemaphoreType.DMA,
        pltpu.SemaphoreType.DMA,
    ],
)
def _gather_kernel(
    w_hbm, idx_hbm, out_hbm, idx_v,
    b0, b1, b2, b3, b4, b5,
    gs0, gs1, gs2, gs3, gs4, gs5,
    ss0, ss1, ss2, ss3, ss4, ss5,
):
    wid = lax.axis_index("s") * NC + lax.axis_index("c")
    base = wid * ROWS_PER_W
    pltpu.sync_copy(idx_hbm.at[pl.ds(wid * ROWS_PER_W, ROWS_PER_W)], idx_v)
    bufs = (b0, b1, b2, b3, b4, b5)
    gsems = (gs0, gs1, gs2, gs3, gs4, gs5)
    ssems = (ss0, ss1, ss2, ss3, ss4, ss5)

    def gather(j):
        slot = j % NBUF
        return pltpu.async_copy(
            w_hbm.at[idx_v.at[pl.ds(j * CHUNK, CHUNK)]], bufs[slot], gsems[slot]
        )

    def scatter(j):
        slot = j % NBUF
        return pltpu.async_copy(
            bufs[slot], out_hbm.at[pl.ds(base + j * CHUNK, CHUNK)], ssems[slot]
        )

    g = [None] * NCHUNK
    s = [None] * NCHUNK
    waited = [False] * NCHUNK
    for j in range(min(NBUF - 1, NCHUNK)):
        g[j] = gather(j)
    for j in range(NCHUNK):
        if j + NBUF - 1 < NCHUNK:
            if j >= 1:
                s[j - 1].wait()
                waited[j - 1] = True
            g[j + NBUF - 1] = gather(j + NBUF - 1)
        g[j].wait()
        s[j] = scatter(j)
    for j in range(NCHUNK):
        if not waited[j]:
            s[j].wait()


def kernel(idx, W):
    w_pad = jnp.pad(W, ((0, 0), (0, VPAD - VOCAB)))
    flat = idx.reshape(N).astype(jnp.int32)
    out = _gather_kernel(w_pad, flat)
    return out[:, :VOCAB].reshape(BATCH, BLOCK, VOCAB)
